# baseline (speedup 1.0000x reference)
import functools

import jax
import jax.numpy as jnp
from jax import lax
from jax.experimental import pallas as pl
from jax.experimental.pallas import tpu as pltpu
from jax.experimental.pallas import tpu_sc as plsc

N_NODES = 10000
N_EDGES = 320000
D_FEAT = 128

NC = 2
NS = 16
L = 16
NW = NC * NS
EPW = N_EDGES // NW
CHUNK = 128
NGROUP = CHUNK // L
NFULL = EPW // CHUNK
TAIL = EPW - NFULL * CHUNK
NORM_BLK = 2000


def _normalize_body(hf_ref, hs_ref, o_ref):
    i = pl.program_id(0)
    nf = N_NODES // NORM_BLK

    @pl.when(i < nf)
    def _():
        h = hf_ref[...]
        o_ref[...] = h * lax.rsqrt(jnp.sum(h * h, axis=1, keepdims=True))

    @pl.when(i >= nf)
    def _():
        h = hs_ref[...]
        o_ref[...] = h * lax.rsqrt(jnp.sum(h * h, axis=1, keepdims=True))


def _normalize_cat(h_first, h_second):
    nf = N_NODES // NORM_BLK
    return pl.pallas_call(
        _normalize_body,
        grid=(2 * nf,),
        in_specs=[
            pl.BlockSpec((NORM_BLK, D_FEAT), lambda i: (jnp.minimum(i, nf - 1), 0)),
            pl.BlockSpec((NORM_BLK, D_FEAT), lambda i: (jnp.maximum(i - nf, 0), 0)),
        ],
        out_specs=pl.BlockSpec((NORM_BLK, D_FEAT), lambda i: (i, 0)),
        out_shape=jax.ShapeDtypeStruct((2 * N_NODES, D_FEAT), jnp.float32),
    )(h_first, h_second)


def _sc_body(ic_hbm, hcat_hbm, out_hbm,
             i_all, out_all, stage, rb0, rb1, sem0, sem1):
    wid = lax.axis_index("s") * NC + lax.axis_index("c")
    base = wid * EPW
    pltpu.sync_copy(ic_hbm.at[pl.ds(2 * base, 2 * EPW)], i_all)

    lanes = lax.iota(jnp.int32, L)

    def issue(c, rbuf, sem):
        idx_lo = i_all.at[pl.ds(2 * c * CHUNK, CHUNK)]
        idx_hi = i_all.at[pl.ds(2 * c * CHUNK + CHUNK, CHUNK)]
        pltpu.async_copy(hcat_hbm.at[idx_lo], rbuf.at[pl.ds(0, CHUNK), :], sem)
        pltpu.async_copy(hcat_hbm.at[idx_hi], rbuf.at[pl.ds(CHUNK, CHUNK), :], sem)

    def drain(c, rbuf, sem):
        idx_lo = i_all.at[pl.ds(2 * c * CHUNK, CHUNK)]
        idx_hi = i_all.at[pl.ds(2 * c * CHUNK + CHUNK, CHUNK)]
        pltpu.make_async_copy(hcat_hbm.at[idx_lo], rbuf.at[pl.ds(0, CHUNK), :], sem).wait()
        pltpu.make_async_copy(hcat_hbm.at[idx_hi], rbuf.at[pl.ds(CHUNK, CHUNK), :], sem).wait()

    def edge_partial(rbuf, l):
        ea = 2 * l
        eb = 2 * l + 1
        acc0 = rbuf[ea, pl.ds(0, L)] * rbuf[eb, pl.ds(0, L)]
        acc1 = rbuf[ea, pl.ds(L, L)] * rbuf[eb, pl.ds(L, L)]
        for j in range(2 * L, D_FEAT, 2 * L):
            acc0 = acc0 + rbuf[ea, pl.ds(j, L)] * rbuf[eb, pl.ds(j, L)]
            acc1 = acc1 + rbuf[ea, pl.ds(j + L, L)] * rbuf[eb, pl.ds(j + L, L)]
        return acc0 + acc1

    def group16(rbuf, e0, out_off):
        for k in range(L):
            s = edge_partial(rbuf, e0 + k)
            col = jnp.full((L,), k, jnp.int32)
            plsc.store_scatter(stage, [lanes, col], s)
        parts = [stage[j, pl.ds(0, L)] for j in range(L)]
        while len(parts) > 1:
            parts = [parts[i] + parts[i + 1] for i in range(0, len(parts), 2)]
        out_all[pl.ds(out_off, L)] = parts[0]

    def compute(c, rbuf):
        def group_body(g, carry2):
            group16(rbuf, g * L, c * CHUNK + g * L)
            return carry2

        lax.fori_loop(0, NGROUP, group_body, 0)

    issue(0, rb0, sem0)
    issue(1, rb1, sem1)

    def pair_body(cc, carry):
        c0 = 2 * cc
        drain(c0, rb0, sem0)
        compute(c0, rb0)

        @pl.when(c0 + 2 < NFULL)
        def _():
            issue(c0 + 2, rb0, sem0)

        drain(c0 + 1, rb1, sem1)
        compute(c0 + 1, rb1)

        @pl.when(c0 + 3 < NFULL)
        def _():
            issue(c0 + 3, rb1, sem1)

        return carry

    lax.fori_loop(0, NFULL // 2, pair_body, 0)

    tail_dst = rb0.at[pl.ds(0, 2 * TAIL), :]
    tail_idx = i_all.at[pl.ds(2 * NFULL * CHUNK, 2 * TAIL)]
    pltpu.async_copy(hcat_hbm.at[tail_idx], tail_dst, sem0).wait()
    group16(rb0, 0, NFULL * CHUNK)

    pltpu.sync_copy(out_all, out_hbm.at[pl.ds(base, EPW)])


@functools.partial(jax.jit, static_argnames=())
def _sc_edge_dots(ic, h_cat):
    mesh = plsc.VectorSubcoreMesh(core_axis_name="c", subcore_axis_name="s")
    return pl.kernel(
        _sc_body,
        out_type=jax.ShapeDtypeStruct((N_EDGES,), jnp.float32),
        mesh=mesh,
        compiler_params=pltpu.CompilerParams(needs_layout_passes=False),
        scratch_types=[
            pltpu.VMEM((2 * EPW,), jnp.int32),
            pltpu.VMEM((EPW,), jnp.float32),
            pltpu.VMEM((L, L + 1), jnp.float32),
            pltpu.VMEM((2 * CHUNK, D_FEAT), jnp.float32),
            pltpu.VMEM((2 * CHUNK, D_FEAT), jnp.float32),
            pltpu.SemaphoreType.DMA,
            pltpu.SemaphoreType.DMA,
        ],
    )(ic, h_cat)


def kernel(edges_supervised, h_first, h_second):
    idx0 = edges_supervised[0].astype(jnp.int32)
    idx1 = edges_supervised[1].astype(jnp.int32)
    ic = jnp.stack([idx0, idx1 + N_NODES], axis=1).ravel()
    h_cat = _normalize_cat(h_first, h_second)
    return _sc_edge_dots(ic, h_cat)


# speedup vs baseline: 1.6315x; 1.6315x over previous
import functools

import jax
import jax.numpy as jnp
from jax import lax
from jax.experimental import pallas as pl
from jax.experimental.pallas import tpu as pltpu
from jax.experimental.pallas import tpu_sc as plsc

N_NODES = 10000
N_EDGES = 320000
D_FEAT = 128

NC = 2
NS = 16
L = 16
NW = NC * NS
EPW = N_EDGES // NW
CHUNK = 80
NGROUP = CHUNK // L
NCHUNK = EPW // CHUNK
NBUF = 3


def _normalize_body(hf_ref, hs_ref, of_ref, os_ref):
    hf = hf_ref[...]
    hs = hs_ref[...]
    of_ref[...] = hf * lax.rsqrt(jnp.sum(hf * hf, axis=1, keepdims=True))
    os_ref[...] = hs * lax.rsqrt(jnp.sum(hs * hs, axis=1, keepdims=True))


def _normalize(h_first, h_second):
    rows = h_first.shape[0]
    blk = 2000
    grid = rows // blk
    spec = pl.BlockSpec((blk, D_FEAT), lambda i: (i, 0))
    return pl.pallas_call(
        _normalize_body,
        grid=(grid,),
        in_specs=[spec, spec],
        out_specs=[spec, spec],
        out_shape=[
            jax.ShapeDtypeStruct(h_first.shape, jnp.float32),
            jax.ShapeDtypeStruct(h_second.shape, jnp.float32),
        ],
    )(h_first, h_second)


def _sc_body(idx0_hbm, idx1_hbm, hf_hbm, hs_hbm, out_hbm,
             i0_all, i1_all, out_all, stage,
             ra0, rb0, ra1, rb1, ra2, rb2,
             sa0, sb0, sa1, sb1, sa2, sb2):
    wid = lax.axis_index("s") * NC + lax.axis_index("c")
    base = wid * EPW
    pltpu.sync_copy(idx0_hbm.at[pl.ds(base, EPW)], i0_all)
    pltpu.sync_copy(idx1_hbm.at[pl.ds(base, EPW)], i1_all)

    lanes = lax.iota(jnp.int32, L)

    def issue(c, ra, rb, sa, sb):
        ia = i0_all.at[pl.ds(c * CHUNK, CHUNK)]
        ib = i1_all.at[pl.ds(c * CHUNK, CHUNK)]
        pltpu.async_copy(hf_hbm.at[ia], ra, sa)
        pltpu.async_copy(hs_hbm.at[ib], rb, sb)

    def drain(c, ra, rb, sa, sb):
        ia = i0_all.at[pl.ds(c * CHUNK, CHUNK)]
        ib = i1_all.at[pl.ds(c * CHUNK, CHUNK)]
        pltpu.make_async_copy(hf_hbm.at[ia], ra, sa).wait()
        pltpu.make_async_copy(hs_hbm.at[ib], rb, sb).wait()

    def edge_partial(ra, rb, e):
        acc0 = ra[e, pl.ds(0, L)] * rb[e, pl.ds(0, L)]
        acc1 = ra[e, pl.ds(L, L)] * rb[e, pl.ds(L, L)]
        for j in range(2 * L, D_FEAT, 2 * L):
            acc0 = acc0 + ra[e, pl.ds(j, L)] * rb[e, pl.ds(j, L)]
            acc1 = acc1 + ra[e, pl.ds(j + L, L)] * rb[e, pl.ds(j + L, L)]
        return acc0 + acc1

    def compute(c, ra, rb):
        def group_body(g, carry2):
            e0 = g * L
            for k in range(L):
                s = edge_partial(ra, rb, e0 + k)
                col = jnp.full((L,), k, jnp.int32)
                plsc.store_scatter(stage, [lanes, col], s)
            parts = [stage[j, pl.ds(0, L)] for j in range(L)]
            while len(parts) > 1:
                parts = [parts[i] + parts[i + 1] for i in range(0, len(parts), 2)]
            out_all[pl.ds(c * CHUNK + e0, L)] = parts[0]
            return carry2

        lax.fori_loop(0, NGROUP, group_body, 0)

    bufs = [(ra0, rb0, sa0, sb0), (ra1, rb1, sa1, sb1), (ra2, rb2, sa2, sb2)]
    for b in range(NBUF):
        issue(b, *bufs[b])

    def ring_body(cc, carry):
        c0 = NBUF * cc
        for b in range(NBUF):
            c = c0 + b
            drain(c, *bufs[b])
            compute(c, bufs[b][0], bufs[b][1])

            @pl.when(c + NBUF < NCHUNK)
            def _():
                issue(c + NBUF, *bufs[b])

        return carry

    lax.fori_loop(0, NCHUNK // NBUF, ring_body, 0)
    for b in range(NCHUNK % NBUF):
        c = (NCHUNK // NBUF) * NBUF + b
        drain(c, *bufs[b])
        compute(c, bufs[b][0], bufs[b][1])

    pltpu.sync_copy(out_all, out_hbm.at[pl.ds(base, EPW)])


@functools.partial(jax.jit, static_argnames=())
def _sc_edge_dots(idx0, idx1, hf_n, hs_n):
    mesh = plsc.VectorSubcoreMesh(core_axis_name="c", subcore_axis_name="s")
    return pl.kernel(
        _sc_body,
        out_type=jax.ShapeDtypeStruct((N_EDGES,), jnp.float32),
        mesh=mesh,
        compiler_params=pltpu.CompilerParams(needs_layout_passes=False),
        scratch_types=[
            pltpu.VMEM((EPW,), jnp.int32),
            pltpu.VMEM((EPW,), jnp.int32),
            pltpu.VMEM((EPW,), jnp.float32),
            pltpu.VMEM((L, L + 1), jnp.float32),
            pltpu.VMEM((CHUNK, D_FEAT), jnp.float32),
            pltpu.VMEM((CHUNK, D_FEAT), jnp.float32),
            pltpu.VMEM((CHUNK, D_FEAT), jnp.float32),
            pltpu.VMEM((CHUNK, D_FEAT), jnp.float32),
            pltpu.VMEM((CHUNK, D_FEAT), jnp.float32),
            pltpu.VMEM((CHUNK, D_FEAT), jnp.float32),
            pltpu.SemaphoreType.DMA,
            pltpu.SemaphoreType.DMA,
            pltpu.SemaphoreType.DMA,
            pltpu.SemaphoreType.DMA,
            pltpu.SemaphoreType.DMA,
            pltpu.SemaphoreType.DMA,
        ],
    )(idx0, idx1, hf_n, hs_n)


def kernel(edges_supervised, h_first, h_second):
    idx0 = edges_supervised[0].astype(jnp.int32)
    idx1 = edges_supervised[1].astype(jnp.int32)
    hf_n, hs_n = _normalize(h_first, h_second)
    return _sc_edge_dots(idx0, idx1, hf_n, hs_n)
